# baseline (speedup 1.0000x reference)
import functools

import jax
import jax.numpy as jnp
from jax import lax
from jax.experimental import pallas as pl
from jax.experimental.pallas import tpu as pltpu
from jax.experimental.pallas import tpu_sc as plsc

NUM_EMB = 1_000_000
D = 64
DW = D // 2
BATCH = 4096
HIST = 50
TOT = BATCH * HIST

NC, NS = 2, 16
NW = NC * NS
PER_W = TOT // NW
CHUNK = 128
NCHUNK = PER_W // CHUNK

_mesh = plsc.VectorSubcoreMesh(core_axis_name="c", subcore_axis_name="s")


@functools.partial(
    pl.kernel,
    mesh=_mesh,
    out_type=jax.ShapeDtypeStruct((TOT, DW), jnp.int32),
    scratch_types=[
        pltpu.VMEM((NCHUNK, CHUNK), jnp.int32),
        pltpu.VMEM((CHUNK, DW), jnp.int32),
        pltpu.VMEM((CHUNK, DW), jnp.int32),
        pltpu.SemaphoreType.DMA,
        pltpu.SemaphoreType.DMA,
    ],
    compiler_params=pltpu.CompilerParams(use_tc_tiling_on_sc=False),
)
def _emb_lookup(table_hbm, idx_hbm, out_hbm, idx_v, buf0, buf1, sem0, sem1):
    wid = lax.axis_index("s") * NC + lax.axis_index("c")
    base = wid * PER_W
    pltpu.sync_copy(idx_hbm.at[wid], idx_v)

    def body(k, _):
        j0 = 2 * k
        j1 = j0 + 1
        c0 = pltpu.async_copy(table_hbm.at[idx_v.at[j0]], buf0, sem0)
        c1 = pltpu.async_copy(table_hbm.at[idx_v.at[j1]], buf1, sem1)
        c0.wait()
        pltpu.sync_copy(buf0, out_hbm.at[pl.ds(base + j0 * CHUNK, CHUNK)])
        c1.wait()
        pltpu.sync_copy(buf1, out_hbm.at[pl.ds(base + j1 * CHUNK, CHUNK)])
        return 0

    lax.fori_loop(0, NCHUNK // 2, body, 0)


def kernel(x, weight):
    idx = x.astype(jnp.int32).reshape(NW, NCHUNK, CHUNK)
    table_i32 = lax.bitcast_convert_type(
        weight.reshape(NUM_EMB, DW, 2), jnp.int32)
    out = _emb_lookup(table_i32, idx)
    out_bf16 = lax.bitcast_convert_type(out, jnp.bfloat16)
    return out_bf16.reshape(BATCH, HIST, D)


# speedup vs baseline: 1.0086x; 1.0086x over previous
import functools

import jax
import jax.numpy as jnp
from jax import lax
from jax.experimental import pallas as pl
from jax.experimental.pallas import tpu as pltpu
from jax.experimental.pallas import tpu_sc as plsc

NUM_EMB = 1_000_000
D = 64
DW = D // 2
BATCH = 4096
HIST = 50
TOT = BATCH * HIST

NC, NS = 2, 16
NW = NC * NS
PER_W = TOT // NW
CHUNK = 128
NCHUNK = PER_W // CHUNK
GRP = 5
GROUP_ROWS = GRP * CHUNK
NGRP = NCHUNK // GRP
NPAIR = NGRP // 2

_mesh = plsc.VectorSubcoreMesh(core_axis_name="c", subcore_axis_name="s")


@functools.partial(
    pl.kernel,
    mesh=_mesh,
    out_type=jax.ShapeDtypeStruct((TOT, DW), jnp.int32),
    scratch_types=[
        pltpu.VMEM((NCHUNK, CHUNK), jnp.int32),
        pltpu.VMEM((GROUP_ROWS, DW), jnp.int32),
        pltpu.VMEM((GROUP_ROWS, DW), jnp.int32),
        pltpu.SemaphoreType.DMA,
        pltpu.SemaphoreType.DMA,
        pltpu.SemaphoreType.DMA,
        pltpu.SemaphoreType.DMA,
    ],
    compiler_params=pltpu.CompilerParams(use_tc_tiling_on_sc=False),
)
def _emb_lookup(table_hbm, idx_hbm, out_hbm, idx_v, buf_a, buf_b,
                sem_ga, sem_gb, sem_wa, sem_wb):
    wid = lax.axis_index("s") * NC + lax.axis_index("c")
    base = wid * PER_W
    pltpu.sync_copy(idx_hbm.at[wid], idx_v)

    def fire_gathers(g, buf, sem):
        for i in range(GRP):
            pltpu.async_copy(
                table_hbm.at[idx_v.at[g * GRP + i]],
                buf.at[pl.ds(i * CHUNK, CHUNK)],
                sem,
            )

    def drain(buf, sem):
        for i in range(GRP):
            pltpu.make_async_copy(
                table_hbm.at[idx_v.at[i]],
                buf.at[pl.ds(i * CHUNK, CHUNK)],
                sem,
            ).wait()

    def fire_write(g, buf, sem):
        return pltpu.async_copy(
            buf, out_hbm.at[pl.ds(base + g * GROUP_ROWS, GROUP_ROWS)], sem)

    def wait_write(g, buf, sem):
        pltpu.make_async_copy(
            buf, out_hbm.at[pl.ds(base + g * GROUP_ROWS, GROUP_ROWS)],
            sem).wait()

    fire_gathers(0, buf_a, sem_ga)

    def body(t, _):
        g0 = 2 * t
        g1 = g0 + 1
        @pl.when(t > 0)
        def _():
            wait_write(g1 - 2, buf_b, sem_wb)
        fire_gathers(g1, buf_b, sem_gb)
        drain(buf_a, sem_ga)
        fire_write(g0, buf_a, sem_wa)
        @pl.when(t < NPAIR - 1)
        def _():
            wait_write(g0, buf_a, sem_wa)
            fire_gathers(g0 + 2, buf_a, sem_ga)
        drain(buf_b, sem_gb)
        fire_write(g1, buf_b, sem_wb)
        return 0

    lax.fori_loop(0, NPAIR, body, 0)
    wait_write(NGRP - 2, buf_a, sem_wa)
    wait_write(NGRP - 1, buf_b, sem_wb)


def kernel(x, weight):
    idx = x.astype(jnp.int32).reshape(NW, NCHUNK, CHUNK)
    table_i32 = lax.bitcast_convert_type(
        weight.reshape(NUM_EMB, DW, 2), jnp.int32)
    out = _emb_lookup(table_i32, idx)
    out_bf16 = lax.bitcast_convert_type(out, jnp.bfloat16)
    return out_bf16.reshape(BATCH, HIST, D)


# speedup vs baseline: 1.7888x; 1.7735x over previous
import functools

import jax
import jax.numpy as jnp
from jax import lax
from jax.experimental import pallas as pl
from jax.experimental.pallas import tpu as pltpu
from jax.experimental.pallas import tpu_sc as plsc

NUM_EMB = 1_000_000
D = 64
DW = D // 2
BATCH = 4096
HIST = 50
TOT = BATCH * HIST

NC, NS = 2, 16
NW = NC * NS
BPW = BATCH // NW
GRP_B = 4
GRP_R = GRP_B * HIST
NGRP = BPW // GRP_B
NPAIR = NGRP // 2

_mesh = plsc.VectorSubcoreMesh(core_axis_name="c", subcore_axis_name="s")


@functools.partial(
    pl.kernel,
    mesh=_mesh,
    out_type=jax.ShapeDtypeStruct((TOT, D), jnp.int32),
    scratch_types=[
        pltpu.VMEM((BPW, HIST), jnp.int32),
        pltpu.VMEM((GRP_R, DW), jnp.int32),
        pltpu.VMEM((GRP_R, DW), jnp.int32),
        pltpu.VMEM((GRP_R, D), jnp.int32),
        pltpu.VMEM((GRP_R, D), jnp.int32),
        pltpu.SemaphoreType.DMA,
        pltpu.SemaphoreType.DMA,
        pltpu.SemaphoreType.DMA,
        pltpu.SemaphoreType.DMA,
    ],
    compiler_params=pltpu.CompilerParams(use_tc_tiling_on_sc=False),
)
def _emb_lookup(table_hbm, x_hbm, out_hbm, idx_v, gba, gbb, wba, wbb,
                sga, sgb, swa, swb):
    wid = lax.axis_index("s") * NC + lax.axis_index("c")
    b0 = wid * BPW
    r0 = wid * BPW * HIST

    pltpu.sync_copy(x_hbm.at[pl.ds(b0, BPW)], idx_v)

    def fire_gathers(g, gb, sem):
        for b in range(GRP_B):
            pltpu.async_copy(
                table_hbm.at[idx_v.at[g * GRP_B + b]],
                gb.at[pl.ds(b * HIST, HIST)], sem)

    def drain_gathers(gb, sem):
        for b in range(GRP_B):
            pltpu.make_async_copy(
                table_hbm.at[idx_v.at[0]],
                gb.at[pl.ds(b * HIST, HIST)], sem).wait()

    def repack(gb, wb):
        def row_body(r, _):
            for h in range(2):
                w = gb[r, pl.ds(h * 16, 16)]
                wb[r, pl.ds(h * 16, 16)] = w << 16
                wb[r, pl.ds(DW + h * 16, 16)] = w & jnp.int32(-65536)
            return 0
        lax.fori_loop(0, GRP_R, row_body, 0)

    def fire_write(g, wb, sem):
        pltpu.async_copy(
            wb, out_hbm.at[pl.ds(r0 + g * GRP_R, GRP_R)], sem)

    def wait_write(wb, sem):
        pltpu.make_async_copy(
            wb, out_hbm.at[pl.ds(r0, GRP_R)], sem).wait()

    fire_gathers(0, gba, sga)

    def body(t, _):
        g0 = 2 * t
        fire_gathers(g0 + 1, gbb, sgb)
        drain_gathers(gba, sga)

        @pl.when(t > 0)
        def _():
            wait_write(wba, swa)
        repack(gba, wba)
        fire_write(g0, wba, swa)

        @pl.when(t < NPAIR - 1)
        def _():
            fire_gathers(g0 + 2, gba, sga)
        drain_gathers(gbb, sgb)

        @pl.when(t > 0)
        def _():
            wait_write(wbb, swb)
        repack(gbb, wbb)
        fire_write(g0 + 1, wbb, swb)
        return 0

    lax.fori_loop(0, NPAIR, body, 0)
    wait_write(wba, swa)
    wait_write(wbb, swb)


def kernel(x, weight):
    x16 = lax.bitcast_convert_type(weight, jnp.int16)
    lo = x16[:, :DW].astype(jnp.int32) & 0xFFFF
    hi = x16[:, DW:].astype(jnp.int32) << 16
    table_i32 = hi | lo
    out_i32 = _emb_lookup(table_i32, x)
    out_f32 = lax.bitcast_convert_type(out_i32, jnp.float32)
    return out_f32.astype(jnp.bfloat16).reshape(BATCH, HIST, D)
